# baseline (speedup 1.0000x reference)
import functools

import jax
import jax.numpy as jnp
from jax import lax
from jax.experimental import pallas as pl
from jax.experimental.pallas import tpu as pltpu
from jax.experimental.pallas import tpu_sc as plsc

_GROUP = 4
_CH = 128


def _gather_kernel(B, V, D):
    info = plsc.get_sparse_core_info()
    NC, NS, L = info.num_cores, info.num_subcores, info.num_lanes
    NW = NC * NS
    G, CH = _GROUP, _CH
    W = G * D
    assert D == 64 and V % G == 0 and B % (8 * NW) == 0
    b_per_w = B // NW
    n_chunks = b_per_w // CH
    n_streams = CH // 128
    shift = G.bit_length() - 1
    mesh = plsc.VectorSubcoreMesh(core_axis_name="c", subcore_axis_name="s")

    @functools.partial(
        pl.kernel,
        mesh=mesh,
        out_type=jax.ShapeDtypeStruct((B, D), jnp.float32),
        compiler_params=pltpu.CompilerParams(needs_layout_passes=False),
        scratch_types=[
            pltpu.VMEM((b_per_w,), jnp.int32),
            pltpu.VMEM((b_per_w // 128, 128), jnp.int32),
            pltpu.VMEM((CH, W), jnp.float32),
            pltpu.VMEM((CH, D), jnp.float32),
            pltpu.SemaphoreType.DMA,
        ],
    )
    def k(w2_hbm, idx_hbm, out_hbm, idx_v, pack_v, rows_v, out_v, sem):
        wid = lax.axis_index("s") * NC + lax.axis_index("c")
        base = wid * b_per_w
        pltpu.sync_copy(idx_hbm.at[pl.ds(base, b_per_w)], idx_v)
        for r in range(b_per_w // 128):
            row = pack_v.at[r]
            for i in range(128 // L):
                v = idx_v[pl.ds(r * 128 + i * L, L)]
                row[pl.ds(i * L, L)] = lax.shift_right_logical(v, shift)

        iota = lax.iota(jnp.int32, L)
        for ch in range(n_chunks):
            copies = [
                pltpu.make_async_copy(
                    w2_hbm.at[pack_v.at[ch * n_streams + g]],
                    rows_v.at[pl.ds(g * 128, 128)],
                    sem,
                )
                for g in range(n_streams)
            ]
            for cp in copies:
                cp.start()
            for cp in copies:
                cp.wait()

            def body(j0, carry, ch=ch):
                rows16 = j0 * L + iota
                v = idx_v[pl.ds(ch * CH + j0 * L, L)]
                off16 = lax.bitwise_and(v, G - 1) * D
                for c in range(D):
                    got = plsc.load_gather(rows_v, [rows16, off16 + c])
                    plsc.store_scatter(
                        out_v, [rows16, jnp.full((L,), c, jnp.int32)], got
                    )
                return carry

            lax.fori_loop(0, CH // L, body, 0)
            pltpu.sync_copy(out_v, out_hbm.at[pl.ds(base + ch * CH, CH)])

    return k


def kernel(x, weight):
    B = x.shape[0]
    V, D = weight.shape
    k = _gather_kernel(B, V, D)
    w2 = weight.reshape(V // _GROUP, _GROUP * D)
    return k(w2, x.astype(jnp.int32))


# speedup vs baseline: 1.2188x; 1.2188x over previous
import functools

import jax
import jax.numpy as jnp
from jax import lax
from jax.experimental import pallas as pl
from jax.experimental.pallas import tpu as pltpu
from jax.experimental.pallas import tpu_sc as plsc

_CH = 256


def _gather_kernel(B, V, D):
    info = plsc.get_sparse_core_info()
    NC, NS, L = info.num_cores, info.num_subcores, info.num_lanes
    NW = NC * NS
    CH = _CH
    assert D == 64 and B % (8 * NW) == 0
    b_per_w = B // NW
    n_chunks = b_per_w // CH
    n_streams = CH // 128
    mesh = plsc.VectorSubcoreMesh(core_axis_name="c", subcore_axis_name="s")

    @functools.partial(
        pl.kernel,
        mesh=mesh,
        out_type=jax.ShapeDtypeStruct((B, 2 * D), jnp.float32),
        compiler_params=pltpu.CompilerParams(needs_layout_passes=False),
        scratch_types=[
            pltpu.VMEM((b_per_w // 128, 128), jnp.int32),
            pltpu.VMEM((CH, 2 * D), jnp.float32),
            pltpu.SemaphoreType.DMA,
        ],
    )
    def k(wpad_hbm, idx_hbm, out_hbm, idx_v, rows_v, sem):
        wid = lax.axis_index("s") * NC + lax.axis_index("c")
        base = wid * b_per_w
        for r in range(b_per_w // 128):
            pltpu.sync_copy(
                idx_hbm.at[pl.ds(base + r * 128, 128)], idx_v.at[r]
            )
        for ch in range(n_chunks):
            copies = [
                pltpu.make_async_copy(
                    wpad_hbm.at[idx_v.at[ch * n_streams + g]],
                    rows_v.at[pl.ds(g * 128, 128)],
                    sem,
                )
                for g in range(n_streams)
            ]
            for cp in copies:
                cp.start()
            for cp in copies:
                cp.wait()
            pltpu.sync_copy(
                rows_v, out_hbm.at[pl.ds(base + ch * CH, CH)]
            )

    return k


def kernel(x, weight):
    B = x.shape[0]
    V, D = weight.shape
    k = _gather_kernel(B, V, D)
    wpad = jnp.pad(weight, ((0, 0), (0, D)))
    out2 = k(wpad, x.astype(jnp.int32))
    return out2[:, :D]
